# Initial kernel scaffold; baseline (speedup 1.0000x reference)
import jax
import jax.numpy as jnp
from jax.experimental import pallas as pl


def kernel(x, edge_index, W_self_0, W_neigh_0, b_0, W_self_1, W_neigh_1, b_1, W_self_2, W_neigh_2, b_2):
    raise NotImplementedError("write your pallas kernel here")


# speedup vs baseline: 3.1549x; 3.1549x over previous
import functools

import jax
import jax.numpy as jnp
from jax import lax
from jax.experimental import pallas as pl
from jax.experimental.pallas import tpu as pltpu
from jax.experimental.pallas import tpu_sc as plsc

N = 10000
E = 320000
D = 128
NC, NS = 2, 16
NW = NC * NS
C = 128
ET_PAD = 10240
NCHUNK = ET_PAD // C
E_PAD = NW * ET_PAD
ACC_N = 10240
RPT = ACC_N // NS
ZR = 16
COPY_R = 80

BR = 2000


def _segsum_body(p_hbm, idx_hbm, out_hbm, acc, src_vm, dst_vm, rows_vm, zbuf,
                 cbuf, sem):
    cid = lax.axis_index("c")
    sid = lax.axis_index("s")
    tid = cid * NS + sid

    z16 = jnp.zeros((16,), jnp.float32)
    for r in range(ZR):
        for c0 in range(D // 16):
            zbuf[r, pl.ds(c0 * 16, 16)] = z16

    def zloop(i, carry):
        pltpu.sync_copy(zbuf, acc.at[pl.ds(sid * RPT + i * ZR, ZR)])
        return carry
    lax.fori_loop(0, RPT // ZR, zloop, 0)

    pltpu.sync_copy(idx_hbm.at[0, tid], src_vm)
    pltpu.sync_copy(idx_hbm.at[1, tid], dst_vm)

    plsc.subcore_barrier()

    def eloop(j, carry):
        pltpu.async_copy(p_hbm.at[src_vm.at[j]], rows_vm, sem).wait()
        pltpu.sync_copy(rows_vm, acc.at[dst_vm.at[j]], add=True)
        return carry
    lax.fori_loop(0, NCHUNK, eloop, 0)

    plsc.subcore_barrier()

    def oloop(i, carry):
        r0 = sid * RPT + i * COPY_R
        pltpu.sync_copy(acc.at[pl.ds(r0, COPY_R)], cbuf)
        pltpu.sync_copy(cbuf, out_hbm.at[cid, pl.ds(r0, COPY_R)])
        return carry
    lax.fori_loop(0, RPT // COPY_R, oloop, 0)


def _make_segsum():
    mesh = plsc.VectorSubcoreMesh(core_axis_name="c", subcore_axis_name="s")
    return pl.kernel(
        _segsum_body,
        out_type=jax.ShapeDtypeStruct((NC, ACC_N, D), jnp.float32),
        mesh=mesh,
        scratch_types=[
            pltpu.VMEM_SHARED((ACC_N, D), jnp.float32),
            pltpu.VMEM((NCHUNK, C), jnp.int32),
            pltpu.VMEM((NCHUNK, C), jnp.int32),
            pltpu.VMEM((C, D), jnp.float32),
            pltpu.VMEM((ZR, D), jnp.float32),
            pltpu.VMEM((COPY_R, D), jnp.float32),
            pltpu.SemaphoreType.DMA,
        ],
    )


def _tc_layer_body(relu, h_ref, agg_ref, ws_ref, wn_ref, b_ref, out_ref):
    a = agg_ref[...]
    agg = a[0] + a[1]
    r = (jnp.dot(h_ref[...], ws_ref[...], preferred_element_type=jnp.float32)
         + jnp.dot(agg, wn_ref[...], preferred_element_type=jnp.float32)
         + b_ref[...])
    out_ref[...] = jnp.maximum(r, 0.0) if relu else r


def _tc_layer(h, agg, ws, wn, b, relu):
    do = ws.shape[1]
    return pl.pallas_call(
        functools.partial(_tc_layer_body, relu),
        grid=(N // BR,),
        in_specs=[
            pl.BlockSpec((BR, D), lambda i: (i, 0)),
            pl.BlockSpec((NC, BR, D), lambda i: (0, i, 0)),
            pl.BlockSpec(ws.shape, lambda i: (0, 0)),
            pl.BlockSpec(wn.shape, lambda i: (0, 0)),
            pl.BlockSpec((1, do), lambda i: (0, 0)),
        ],
        out_specs=pl.BlockSpec((BR, do), lambda i: (i, 0)),
        out_shape=jax.ShapeDtypeStruct((N, do), jnp.float32),
    )(h, agg, ws, wn, b[None, :])


def kernel(x, edge_index, W_self_0, W_neigh_0, b_0, W_self_1, W_neigh_1, b_1,
           W_self_2, W_neigh_2, b_2):
    pad = E_PAD - E
    src = jnp.concatenate([edge_index[0], jnp.zeros((pad,), jnp.int32)])
    dst = jnp.concatenate([edge_index[1], jnp.full((pad,), N, jnp.int32)])
    eidx = jnp.stack([src, dst]).reshape(2, NW, NCHUNK, C)

    segsum = _make_segsum()

    agg0 = segsum(x, eidx)
    h1 = _tc_layer(x, agg0, W_self_0, W_neigh_0, b_0, relu=True)
    agg1 = segsum(h1, eidx)
    h2 = _tc_layer(h1, agg1, W_self_1, W_neigh_1, b_1, relu=True)
    agg2 = segsum(h2, eidx)
    return _tc_layer(h2, agg2, W_self_2, W_neigh_2, b_2, relu=False)


# speedup vs baseline: 3.4643x; 1.0981x over previous
import functools

import jax
import jax.numpy as jnp
from jax import lax
from jax.experimental import pallas as pl
from jax.experimental.pallas import tpu as pltpu
from jax.experimental.pallas import tpu_sc as plsc

N = 10000
E = 320000
D = 128
NC, NS = 2, 16
NW = NC * NS
C = 128
ET_PAD = 10240
NCHUNK = ET_PAD // C
E_PAD = NW * ET_PAD
ACC_N = 10240
RPT = ACC_N // NS
ZR = 16
COPY_R = 128

NBUF = 2
NHALF = 2
HCHUNK = NCHUNK // NHALF
HG = HCHUNK // NBUF

BR = 2000


def _segsum_body(p_hbm, idx_hbm, out_hbm, acc, srcb, dstb, rows0, rows1,
                 gsem0, gsem1, ssem0, ssem1):
    rows = [rows0, rows1]
    gsem = [gsem0, gsem1]
    ssem = [ssem0, ssem1]
    cid = lax.axis_index("c")
    sid = lax.axis_index("s")
    tid = cid * NS + sid

    z16 = jnp.zeros((16,), jnp.float32)
    for r in range(ZR):
        for c0 in range(D // 16):
            rows0[r, pl.ds(c0 * 16, 16)] = z16

    def zloop(i, carry):
        pltpu.sync_copy(rows0.at[pl.ds(0, ZR)],
                        acc.at[pl.ds(sid * RPT + i * ZR, ZR)])
        return carry
    lax.fori_loop(0, RPT // ZR, zloop, 0)

    plsc.subcore_barrier()

    for h in range(NHALF):
        pltpu.sync_copy(idx_hbm.at[0, tid, pl.ds(h * HCHUNK, HCHUNK)], srcb)
        pltpu.sync_copy(idx_hbm.at[1, tid, pl.ds(h * HCHUNK, HCHUNK)], dstb)

        for b in range(NBUF):
            pltpu.async_copy(p_hbm.at[srcb.at[b]], rows[b], gsem[b])

        def eloop(g, carry):
            for b in range(NBUF):
                j = g * NBUF + b
                pltpu.make_async_copy(p_hbm.at[srcb.at[j]], rows[b],
                                      gsem[b]).wait()
                pltpu.async_copy(rows[b], acc.at[dstb.at[j]], ssem[b],
                                 add=True)
            for b in range(NBUF):
                j = g * NBUF + b

                @pl.when(g < HG - 1)
                def _():
                    pltpu.make_async_copy(rows[b], acc.at[dstb.at[j]],
                                          ssem[b]).wait()
                    pltpu.async_copy(p_hbm.at[srcb.at[j + NBUF]], rows[b],
                                     gsem[b])
            return carry
        lax.fori_loop(0, HG, eloop, 0)

        for b in range(NBUF):
            pltpu.make_async_copy(rows[b],
                                  acc.at[dstb.at[HCHUNK - NBUF + b]],
                                  ssem[b]).wait()

    plsc.subcore_barrier()

    def oloop(i, carry):
        r0 = sid * RPT + i * COPY_R
        pltpu.sync_copy(acc.at[pl.ds(r0, COPY_R)], rows0)
        pltpu.sync_copy(rows0, out_hbm.at[cid, pl.ds(r0, COPY_R)])
        return carry
    lax.fori_loop(0, RPT // COPY_R, oloop, 0)


def _make_segsum():
    mesh = plsc.VectorSubcoreMesh(core_axis_name="c", subcore_axis_name="s")
    return pl.kernel(
        _segsum_body,
        out_type=jax.ShapeDtypeStruct((NC, ACC_N, D), jnp.float32),
        mesh=mesh,
        scratch_types=[
            pltpu.VMEM_SHARED((ACC_N, D), jnp.float32),
            pltpu.VMEM((HCHUNK, C), jnp.int32),
            pltpu.VMEM((HCHUNK, C), jnp.int32),
            pltpu.VMEM((C, D), jnp.float32),
            pltpu.VMEM((C, D), jnp.float32),
            pltpu.SemaphoreType.DMA,
            pltpu.SemaphoreType.DMA,
            pltpu.SemaphoreType.DMA,
            pltpu.SemaphoreType.DMA,
        ],
    )


def _tc_layer_body(relu, h_ref, agg_ref, ws_ref, wn_ref, b_ref, out_ref):
    a = agg_ref[...]
    agg = a[0] + a[1]
    r = (jnp.dot(h_ref[...], ws_ref[...], preferred_element_type=jnp.float32)
         + jnp.dot(agg, wn_ref[...], preferred_element_type=jnp.float32)
         + b_ref[...])
    out_ref[...] = jnp.maximum(r, 0.0) if relu else r


def _tc_layer(h, agg, ws, wn, b, relu):
    do = ws.shape[1]
    return pl.pallas_call(
        functools.partial(_tc_layer_body, relu),
        grid=(N // BR,),
        in_specs=[
            pl.BlockSpec((BR, D), lambda i: (i, 0)),
            pl.BlockSpec((NC, BR, D), lambda i: (0, i, 0)),
            pl.BlockSpec(ws.shape, lambda i: (0, 0)),
            pl.BlockSpec(wn.shape, lambda i: (0, 0)),
            pl.BlockSpec((1, do), lambda i: (0, 0)),
        ],
        out_specs=pl.BlockSpec((BR, do), lambda i: (i, 0)),
        out_shape=jax.ShapeDtypeStruct((N, do), jnp.float32),
    )(h, agg, ws, wn, b[None, :])


def kernel(x, edge_index, W_self_0, W_neigh_0, b_0, W_self_1, W_neigh_1, b_1,
           W_self_2, W_neigh_2, b_2):
    pad = E_PAD - E
    src = jnp.concatenate([edge_index[0], jnp.zeros((pad,), jnp.int32)])
    dst = jnp.concatenate([edge_index[1], jnp.full((pad,), N, jnp.int32)])
    eidx = jnp.stack([src, dst]).reshape(2, NW, NCHUNK, C)

    segsum = _make_segsum()

    agg0 = segsum(x, eidx)
    h1 = _tc_layer(x, agg0, W_self_0, W_neigh_0, b_0, relu=True)
    agg1 = segsum(h1, eidx)
    h2 = _tc_layer(h1, agg1, W_self_1, W_neigh_1, b_1, relu=True)
    agg2 = segsum(h2, eidx)
    return _tc_layer(h2, agg2, W_self_2, W_neigh_2, b_2, relu=False)


# speedup vs baseline: 3.4785x; 1.0041x over previous
import functools

import jax
import jax.numpy as jnp
from jax import lax
from jax.experimental import pallas as pl
from jax.experimental.pallas import tpu as pltpu
from jax.experimental.pallas import tpu_sc as plsc

N = 10000
E = 320000
D = 128
NC, NS = 2, 16
NW = NC * NS
C = 128
ET_PAD = 10240
NCHUNK = ET_PAD // C
E_PAD = NW * ET_PAD
ACC_N = 10240
RPT = ACC_N // NS
ZR = 16
COPY_R = 128

NBUF = 2
NHALF = 2
HCHUNK = NCHUNK // NHALF
HG = HCHUNK // NBUF

BR = 2000


def _segsum_body(p_hbm, idx_hbm, out_hbm, acc, idxb, rows0, rows1,
                 gsem0, gsem1, ssem0, ssem1):
    rows = [rows0, rows1]
    gsem = [gsem0, gsem1]
    ssem = [ssem0, ssem1]
    cid = lax.axis_index("c")
    sid = lax.axis_index("s")
    tid = cid * NS + sid
    nz = RPT // COPY_R

    z16 = jnp.zeros((16,), jnp.float32)
    for r in range(C):
        for c0 in range(D // 16):
            rows0[r, pl.ds(c0 * 16, 16)] = z16
    for i in range(nz):
        pltpu.async_copy(rows0, acc.at[pl.ds(sid * RPT + i * COPY_R, COPY_R)],
                         gsem0)
    pltpu.async_copy(idx_hbm.at[tid, pl.ds(0, HCHUNK)], idxb, gsem1)
    for i in range(nz):
        pltpu.make_async_copy(
            rows0, acc.at[pl.ds(sid * RPT + i * COPY_R, COPY_R)],
            gsem0).wait()
    pltpu.make_async_copy(idx_hbm.at[tid, pl.ds(0, HCHUNK)], idxb,
                          gsem1).wait()

    plsc.subcore_barrier()

    for h in range(NHALF):
        if h > 0:
            pltpu.sync_copy(idx_hbm.at[tid, pl.ds(h * HCHUNK, HCHUNK)], idxb)

        for b in range(NBUF):
            pltpu.async_copy(p_hbm.at[idxb.at[b, 0]], rows[b], gsem[b])

        def eloop(g, carry):
            for b in range(NBUF):
                j = g * NBUF + b
                pltpu.make_async_copy(p_hbm.at[idxb.at[j, 0]], rows[b],
                                      gsem[b]).wait()
                pltpu.async_copy(rows[b], acc.at[idxb.at[j, 1]], ssem[b],
                                 add=True)
            for b in range(NBUF):
                j = g * NBUF + b

                @pl.when(g < HG - 1)
                def _():
                    pltpu.make_async_copy(rows[b], acc.at[idxb.at[j, 1]],
                                          ssem[b]).wait()
                    pltpu.async_copy(p_hbm.at[idxb.at[j + NBUF, 0]], rows[b],
                                     gsem[b])
            return carry
        lax.fori_loop(0, HG, eloop, 0)

        for b in range(NBUF):
            pltpu.make_async_copy(rows[b],
                                  acc.at[idxb.at[HCHUNK - NBUF + b, 1]],
                                  ssem[b]).wait()

    plsc.subcore_barrier()

    for i in range(nz):
        b = i % NBUF
        r0 = sid * RPT + i * COPY_R
        if i >= NBUF:
            rp = sid * RPT + (i - NBUF) * COPY_R
            pltpu.make_async_copy(
                rows[b], out_hbm.at[cid, pl.ds(rp, COPY_R)], ssem[b]).wait()
        pltpu.async_copy(acc.at[pl.ds(r0, COPY_R)], rows[b], gsem[b])
        pltpu.make_async_copy(acc.at[pl.ds(r0, COPY_R)], rows[b],
                              gsem[b]).wait()
        pltpu.async_copy(rows[b], out_hbm.at[cid, pl.ds(r0, COPY_R)], ssem[b])
    for i in range(nz - NBUF, nz):
        b = i % NBUF
        r0 = sid * RPT + i * COPY_R
        pltpu.make_async_copy(rows[b], out_hbm.at[cid, pl.ds(r0, COPY_R)],
                              ssem[b]).wait()


def _make_segsum():
    mesh = plsc.VectorSubcoreMesh(core_axis_name="c", subcore_axis_name="s")
    return pl.kernel(
        _segsum_body,
        out_type=jax.ShapeDtypeStruct((NC, ACC_N, D), jnp.float32),
        mesh=mesh,
        scratch_types=[
            pltpu.VMEM_SHARED((ACC_N, D), jnp.float32),
            pltpu.VMEM((HCHUNK, 2, C), jnp.int32),
            pltpu.VMEM((C, D), jnp.float32),
            pltpu.VMEM((C, D), jnp.float32),
            pltpu.SemaphoreType.DMA,
            pltpu.SemaphoreType.DMA,
            pltpu.SemaphoreType.DMA,
            pltpu.SemaphoreType.DMA,
        ],
    )


def _tc_layer_body(relu, h_ref, agg_ref, ws_ref, wn_ref, b_ref, out_ref):
    a = agg_ref[...]
    agg = a[0] + a[1]
    r = (jnp.dot(h_ref[...], ws_ref[...], preferred_element_type=jnp.float32)
         + jnp.dot(agg, wn_ref[...], preferred_element_type=jnp.float32)
         + b_ref[...])
    out_ref[...] = jnp.maximum(r, 0.0) if relu else r


def _tc_layer(h, agg, ws, wn, b, relu):
    do = ws.shape[1]
    return pl.pallas_call(
        functools.partial(_tc_layer_body, relu),
        grid=(N // BR,),
        in_specs=[
            pl.BlockSpec((BR, D), lambda i: (i, 0)),
            pl.BlockSpec((NC, BR, D), lambda i: (0, i, 0)),
            pl.BlockSpec(ws.shape, lambda i: (0, 0)),
            pl.BlockSpec(wn.shape, lambda i: (0, 0)),
            pl.BlockSpec((1, do), lambda i: (0, 0)),
        ],
        out_specs=pl.BlockSpec((BR, do), lambda i: (i, 0)),
        out_shape=jax.ShapeDtypeStruct((N, do), jnp.float32),
    )(h, agg, ws, wn, b[None, :])


def kernel(x, edge_index, W_self_0, W_neigh_0, b_0, W_self_1, W_neigh_1, b_1,
           W_self_2, W_neigh_2, b_2):
    pad = E_PAD - E
    src = jnp.concatenate([edge_index[0], jnp.zeros((pad,), jnp.int32)])
    dst = jnp.concatenate([edge_index[1], jnp.full((pad,), N, jnp.int32)])
    eidx = jnp.stack([src.reshape(NW, NCHUNK, C),
                      dst.reshape(NW, NCHUNK, C)], axis=2)

    segsum = _make_segsum()

    agg0 = segsum(x, eidx)
    h1 = _tc_layer(x, agg0, W_self_0, W_neigh_0, b_0, relu=True)
    agg1 = segsum(h1, eidx)
    h2 = _tc_layer(h1, agg1, W_self_1, W_neigh_1, b_1, relu=True)
    agg2 = segsum(h2, eidx)
    return _tc_layer(h2, agg2, W_self_2, W_neigh_2, b_2, relu=False)


# speedup vs baseline: 3.8185x; 1.0977x over previous
import functools

import jax
import jax.numpy as jnp
from jax import lax
from jax.experimental import pallas as pl
from jax.experimental.pallas import tpu as pltpu
from jax.experimental.pallas import tpu_sc as plsc

N = 10000
E = 320000
D = 128
NC, NS = 2, 16
NW = NC * NS
C = 128
ET_PAD = 10240
NCHUNK = ET_PAD // C
E_PAD = NW * ET_PAD
ACC_N = 10240
RPT = ACC_N // NS
ZR = 16
COPY_R = 128

NBUF = 2
HCHUNK = 40
HG = HCHUNK // NBUF
TOTAL_CHUNKS = E_PAD // C
NPASS_A, NPASS_B = 3, 1
CH_A, CH_B = NPASS_A * HCHUNK, NPASS_B * HCHUNK

BR = 2000


def _segsum_body(p_hbm, idx_hbm, out_hbm, acc, idxb, rows0, rows1,
                 gsem0, gsem1, ssem0, ssem1):
    rows = [rows0, rows1]
    gsem = [gsem0, gsem1]
    ssem = [ssem0, ssem1]
    cid = lax.axis_index("c")
    sid = lax.axis_index("s")
    nz = RPT // COPY_R
    chunk_base = jnp.where(cid == 0, sid * CH_A, NS * CH_A + sid * CH_B)
    npass = jnp.where(cid == 0, NPASS_A, NPASS_B)

    z16 = jnp.zeros((16,), jnp.float32)
    for r in range(C):
        for c0 in range(D // 16):
            rows0[r, pl.ds(c0 * 16, 16)] = z16
    for i in range(nz):
        pltpu.async_copy(rows0, acc.at[pl.ds(sid * RPT + i * COPY_R, COPY_R)],
                         gsem0)
    pltpu.async_copy(idx_hbm.at[pl.ds(chunk_base, HCHUNK)], idxb, gsem1)
    for i in range(nz):
        pltpu.make_async_copy(
            rows0, acc.at[pl.ds(sid * RPT + i * COPY_R, COPY_R)],
            gsem0).wait()
    pltpu.make_async_copy(idx_hbm.at[pl.ds(chunk_base, HCHUNK)], idxb,
                          gsem1).wait()

    plsc.subcore_barrier()

    def ploop(h, pcarry):
        @pl.when(h > 0)
        def _():
            pltpu.sync_copy(idx_hbm.at[pl.ds(chunk_base + h * HCHUNK,
                                             HCHUNK)], idxb)

        for b in range(NBUF):
            pltpu.async_copy(p_hbm.at[idxb.at[b, 0]], rows[b], gsem[b])

        def eloop(g, carry):
            for b in range(NBUF):
                j = g * NBUF + b
                pltpu.make_async_copy(p_hbm.at[idxb.at[j, 0]], rows[b],
                                      gsem[b]).wait()
                pltpu.async_copy(rows[b], acc.at[idxb.at[j, 1]], ssem[b],
                                 add=True)
            for b in range(NBUF):
                j = g * NBUF + b

                @pl.when(g < HG - 1)
                def _():
                    pltpu.make_async_copy(rows[b], acc.at[idxb.at[j, 1]],
                                          ssem[b]).wait()
                    pltpu.async_copy(p_hbm.at[idxb.at[j + NBUF, 0]], rows[b],
                                     gsem[b])
            return carry
        lax.fori_loop(0, HG, eloop, 0)

        for b in range(NBUF):
            pltpu.make_async_copy(rows[b],
                                  acc.at[idxb.at[HCHUNK - NBUF + b, 1]],
                                  ssem[b]).wait()
        return pcarry
    lax.fori_loop(0, npass, ploop, 0)

    plsc.subcore_barrier()

    for i in range(nz):
        b = i % NBUF
        r0 = sid * RPT + i * COPY_R
        if i >= NBUF:
            rp = sid * RPT + (i - NBUF) * COPY_R
            pltpu.make_async_copy(
                rows[b], out_hbm.at[cid, pl.ds(rp, COPY_R)], ssem[b]).wait()
        pltpu.async_copy(acc.at[pl.ds(r0, COPY_R)], rows[b], gsem[b])
        pltpu.make_async_copy(acc.at[pl.ds(r0, COPY_R)], rows[b],
                              gsem[b]).wait()
        pltpu.async_copy(rows[b], out_hbm.at[cid, pl.ds(r0, COPY_R)], ssem[b])
    for i in range(nz - NBUF, nz):
        b = i % NBUF
        r0 = sid * RPT + i * COPY_R
        pltpu.make_async_copy(rows[b], out_hbm.at[cid, pl.ds(r0, COPY_R)],
                              ssem[b]).wait()


def _make_segsum():
    mesh = plsc.VectorSubcoreMesh(core_axis_name="c", subcore_axis_name="s")
    return pl.kernel(
        _segsum_body,
        out_type=jax.ShapeDtypeStruct((NC, ACC_N, D), jnp.float32),
        mesh=mesh,
        scratch_types=[
            pltpu.VMEM_SHARED((ACC_N, D), jnp.float32),
            pltpu.VMEM((HCHUNK, 2, C), jnp.int32),
            pltpu.VMEM((C, D), jnp.float32),
            pltpu.VMEM((C, D), jnp.float32),
            pltpu.SemaphoreType.DMA,
            pltpu.SemaphoreType.DMA,
            pltpu.SemaphoreType.DMA,
            pltpu.SemaphoreType.DMA,
        ],
    )


def _tc_layer_body(relu, h_ref, agg_ref, ws_ref, wn_ref, b_ref, out_ref):
    a = agg_ref[...]
    agg = a[0] + a[1]
    r = (jnp.dot(h_ref[...], ws_ref[...], preferred_element_type=jnp.float32)
         + jnp.dot(agg, wn_ref[...], preferred_element_type=jnp.float32)
         + b_ref[...])
    out_ref[...] = jnp.maximum(r, 0.0) if relu else r


def _tc_layer(h, agg, ws, wn, b, relu):
    do = ws.shape[1]
    return pl.pallas_call(
        functools.partial(_tc_layer_body, relu),
        grid=(N // BR,),
        in_specs=[
            pl.BlockSpec((BR, D), lambda i: (i, 0)),
            pl.BlockSpec((NC, BR, D), lambda i: (0, i, 0)),
            pl.BlockSpec(ws.shape, lambda i: (0, 0)),
            pl.BlockSpec(wn.shape, lambda i: (0, 0)),
            pl.BlockSpec((1, do), lambda i: (0, 0)),
        ],
        out_specs=pl.BlockSpec((BR, do), lambda i: (i, 0)),
        out_shape=jax.ShapeDtypeStruct((N, do), jnp.float32),
    )(h, agg, ws, wn, b[None, :])


def kernel(x, edge_index, W_self_0, W_neigh_0, b_0, W_self_1, W_neigh_1, b_1,
           W_self_2, W_neigh_2, b_2):
    pad = E_PAD - E
    src = jnp.concatenate([edge_index[0], jnp.zeros((pad,), jnp.int32)])
    dst = jnp.concatenate([edge_index[1], jnp.full((pad,), N, jnp.int32)])
    eidx = jnp.stack([src.reshape(TOTAL_CHUNKS, C),
                      dst.reshape(TOTAL_CHUNKS, C)], axis=1)

    segsum = _make_segsum()

    agg0 = segsum(x, eidx)
    h1 = _tc_layer(x, agg0, W_self_0, W_neigh_0, b_0, relu=True)
    agg1 = segsum(h1, eidx)
    h2 = _tc_layer(h1, agg1, W_self_1, W_neigh_1, b_1, relu=True)
    agg2 = segsum(h2, eidx)
    return _tc_layer(h2, agg2, W_self_2, W_neigh_2, b_2, relu=False)


# speedup vs baseline: 3.8213x; 1.0007x over previous
import functools

import jax
import jax.numpy as jnp
from jax import lax
from jax.experimental import pallas as pl
from jax.experimental.pallas import tpu as pltpu
from jax.experimental.pallas import tpu_sc as plsc

N = 10000
E = 320000
D = 128
NC, NS = 2, 16
NW = NC * NS
C = 128
ET_PAD = 10240
NCHUNK = ET_PAD // C
E_PAD = NW * ET_PAD
ACC_N = 10240
RPT = ACC_N // NS
ZR = 16
COPY_R = 128

NBUF = 2
HCHUNK = 40
HG = HCHUNK // NBUF
TOTAL_CHUNKS = E_PAD // C
NPASS_A, NPASS_B = 3, 1
CH_A, CH_B = NPASS_A * HCHUNK, NPASS_B * HCHUNK

BR = 2000


def _segsum_body(p_hbm, idx_hbm, out_hbm, acc, idxb, rows0, rows1,
                 gsem0, gsem1, ssem0, ssem1):
    rows = [rows0, rows1]
    gsem = [gsem0, gsem1]
    ssem = [ssem0, ssem1]
    cid = lax.axis_index("c")
    sid = lax.axis_index("s")
    nz = RPT // COPY_R
    chunk_base = jnp.where(cid == 0, sid * CH_A, NS * CH_A + sid * CH_B)
    npass = jnp.where(cid == 0, NPASS_A, NPASS_B)

    _s1 = jax.named_scope("sg_zero"); _s1.__enter__()
    z16 = jnp.zeros((16,), jnp.float32)
    for r in range(C):
        for c0 in range(D // 16):
            rows0[r, pl.ds(c0 * 16, 16)] = z16
    for i in range(nz):
        pltpu.async_copy(rows0, acc.at[pl.ds(sid * RPT + i * COPY_R, COPY_R)],
                         gsem0)
    pltpu.async_copy(idx_hbm.at[pl.ds(chunk_base, HCHUNK)], idxb, gsem1)
    for i in range(nz):
        pltpu.make_async_copy(
            rows0, acc.at[pl.ds(sid * RPT + i * COPY_R, COPY_R)],
            gsem0).wait()
    pltpu.make_async_copy(idx_hbm.at[pl.ds(chunk_base, HCHUNK)], idxb,
                          gsem1).wait()

    _s1.__exit__(None, None, None)
    _s2 = jax.named_scope("sg_bar1"); _s2.__enter__()
    plsc.subcore_barrier()
    _s2.__exit__(None, None, None)

    _s3 = jax.named_scope("sg_edges"); _s3.__enter__()
    def ploop(h, pcarry):
        @pl.when(h > 0)
        def _():
            pltpu.sync_copy(idx_hbm.at[pl.ds(chunk_base + h * HCHUNK,
                                             HCHUNK)], idxb)

        for b in range(NBUF):
            pltpu.async_copy(p_hbm.at[idxb.at[b, 0]], rows[b], gsem[b])

        def eloop(g, carry):
            for b in range(NBUF):
                j = g * NBUF + b
                pltpu.make_async_copy(p_hbm.at[idxb.at[j, 0]], rows[b],
                                      gsem[b]).wait()
                pltpu.async_copy(rows[b], acc.at[idxb.at[j, 1]], ssem[b],
                                 add=True)
            for b in range(NBUF):
                j = g * NBUF + b

                @pl.when(g < HG - 1)
                def _():
                    pltpu.make_async_copy(rows[b], acc.at[idxb.at[j, 1]],
                                          ssem[b]).wait()
                    pltpu.async_copy(p_hbm.at[idxb.at[j + NBUF, 0]], rows[b],
                                     gsem[b])
            return carry
        lax.fori_loop(0, HG, eloop, 0)

        for b in range(NBUF):
            pltpu.make_async_copy(rows[b],
                                  acc.at[idxb.at[HCHUNK - NBUF + b, 1]],
                                  ssem[b]).wait()
        return pcarry
    lax.fori_loop(0, npass, ploop, 0)
    _s3.__exit__(None, None, None)

    _s4 = jax.named_scope("sg_bar2"); _s4.__enter__()
    plsc.subcore_barrier()
    _s4.__exit__(None, None, None)

    _s5 = jax.named_scope("sg_copyout"); _s5.__enter__()
    for i in range(nz):
        b = i % NBUF
        r0 = sid * RPT + i * COPY_R
        if i >= NBUF:
            rp = sid * RPT + (i - NBUF) * COPY_R
            pltpu.make_async_copy(
                rows[b], out_hbm.at[cid, pl.ds(rp, COPY_R)], ssem[b]).wait()
        pltpu.async_copy(acc.at[pl.ds(r0, COPY_R)], rows[b], gsem[b])
        pltpu.make_async_copy(acc.at[pl.ds(r0, COPY_R)], rows[b],
                              gsem[b]).wait()
        pltpu.async_copy(rows[b], out_hbm.at[cid, pl.ds(r0, COPY_R)], ssem[b])
    for i in range(nz - NBUF, nz):
        b = i % NBUF
        r0 = sid * RPT + i * COPY_R
        pltpu.make_async_copy(rows[b], out_hbm.at[cid, pl.ds(r0, COPY_R)],
                              ssem[b]).wait()
    _s5.__exit__(None, None, None)


def _make_segsum():
    mesh = plsc.VectorSubcoreMesh(core_axis_name="c", subcore_axis_name="s")
    return pl.kernel(
        _segsum_body,
        out_type=jax.ShapeDtypeStruct((NC, ACC_N, D), jnp.float32),
        mesh=mesh,
        scratch_types=[
            pltpu.VMEM_SHARED((ACC_N, D), jnp.float32),
            pltpu.VMEM((HCHUNK, 2, C), jnp.int32),
            pltpu.VMEM((C, D), jnp.float32),
            pltpu.VMEM((C, D), jnp.float32),
            pltpu.SemaphoreType.DMA,
            pltpu.SemaphoreType.DMA,
            pltpu.SemaphoreType.DMA,
            pltpu.SemaphoreType.DMA,
        ],
    )


def _tc_layer_body(relu, h_ref, agg_ref, ws_ref, wn_ref, b_ref, out_ref):
    a = agg_ref[...]
    agg = a[0] + a[1]
    r = (jnp.dot(h_ref[...], ws_ref[...], preferred_element_type=jnp.float32)
         + jnp.dot(agg, wn_ref[...], preferred_element_type=jnp.float32)
         + b_ref[...])
    out_ref[...] = jnp.maximum(r, 0.0) if relu else r


def _tc_layer(h, agg, ws, wn, b, relu):
    do = ws.shape[1]
    return pl.pallas_call(
        functools.partial(_tc_layer_body, relu),
        grid=(N // BR,),
        in_specs=[
            pl.BlockSpec((BR, D), lambda i: (i, 0)),
            pl.BlockSpec((NC, BR, D), lambda i: (0, i, 0)),
            pl.BlockSpec(ws.shape, lambda i: (0, 0)),
            pl.BlockSpec(wn.shape, lambda i: (0, 0)),
            pl.BlockSpec((1, do), lambda i: (0, 0)),
        ],
        out_specs=pl.BlockSpec((BR, do), lambda i: (i, 0)),
        out_shape=jax.ShapeDtypeStruct((N, do), jnp.float32),
    )(h, agg, ws, wn, b[None, :])


def kernel(x, edge_index, W_self_0, W_neigh_0, b_0, W_self_1, W_neigh_1, b_1,
           W_self_2, W_neigh_2, b_2):
    pad = E_PAD - E
    src = jnp.concatenate([edge_index[0], jnp.zeros((pad,), jnp.int32)])
    dst = jnp.concatenate([edge_index[1], jnp.full((pad,), N, jnp.int32)])
    eidx = jnp.stack([src.reshape(TOTAL_CHUNKS, C),
                      dst.reshape(TOTAL_CHUNKS, C)], axis=1)

    segsum = _make_segsum()

    agg0 = segsum(x, eidx)
    h1 = _tc_layer(x, agg0, W_self_0, W_neigh_0, b_0, relu=True)
    agg1 = segsum(h1, eidx)
    h2 = _tc_layer(h1, agg1, W_self_1, W_neigh_1, b_1, relu=True)
    agg2 = segsum(h2, eidx)
    return _tc_layer(h2, agg2, W_self_2, W_neigh_2, b_2, relu=False)
